# Initial kernel scaffold; baseline (speedup 1.0000x reference)
import jax
import jax.numpy as jnp
from jax.experimental import pallas as pl


def kernel(embed, shortlist, weight, bias):
    raise NotImplementedError("write your pallas kernel here")


# speedup vs baseline: 12.4941x; 12.4941x over previous
import jax
import jax.numpy as jnp
from jax import lax
from jax.experimental import pallas as pl
from jax.experimental.pallas import tpu as pltpu
from jax.experimental.pallas import tpu_sc as plsc

B, L, D, V = 4096, 200, 512, 100000
NC, NS, LANES = 2, 16, 16
NW = NC * NS
BPW = B // NW
LC = 96
NG = LC // LANES
LT = L - 2 * LC
LPAD = 256


def _sc_body(embed_hbm, sl_hbm, w_hbm, bias_hbm, out_hbm,
             emb_v, idx_v, rows_v, rowst_v, out_v, sems):
    wid = lax.axis_index("s") * NC + lax.axis_index("c")
    b0 = wid * BPW
    lane = lax.iota(jnp.int32, LANES)

    def row_body(r, _):
        b = b0 + r
        pltpu.sync_copy(embed_hbm.at[b], emb_v)
        pltpu.sync_copy(sl_hbm.at[b], idx_v)
        copies = []
        for c in range(2):
            idx = idx_v.at[pl.ds(c * LC, LC)]
            copies.append(pltpu.async_copy(
                w_hbm.at[idx], rows_v.at[c], sems.at[c]))
            copies.append(pltpu.async_copy(
                bias_hbm.at[idx], out_v.at[pl.ds(c * LC, LC)], sems.at[c]))
        idxt = idx_v.at[pl.ds(2 * LC, LT)]
        copies.append(pltpu.async_copy(
            w_hbm.at[idxt], rowst_v.at[pl.ds(0, LT)], sems.at[2]))
        copies.append(pltpu.async_copy(
            bias_hbm.at[idxt], out_v.at[pl.ds(2 * LC, LT)], sems.at[2]))

        e = [emb_v[pl.ds(j * LANES, LANES)] for j in range(D // LANES)]

        dn = lax.GatherDimensionNumbers(offset_dims=(), collapsed_slice_dims=(0,),
                                        start_index_map=(0,))
        bfly_idx = [lane ^ k for k in (1, 2, 4, 8)]

        def lanesum(acc):
            for idx in bfly_idx:
                acc = acc + lax.gather(acc, idx[:, None], dn, (1,),
                                       mode=lax.GatherScatterMode.PROMISE_IN_BOUNDS)
            return acc

        def dot16(rows_ref, lbase, bufsel):
            vec = jnp.zeros((LANES,), jnp.float32)
            for i in range(LANES):
                l = lbase + i
                if bufsel is None:
                    acc = e[0] * rows_ref[l, pl.ds(0, LANES)]
                    for j in range(1, D // LANES):
                        acc = acc + e[j] * rows_ref[l, pl.ds(j * LANES, LANES)]
                else:
                    acc = e[0] * rows_ref[bufsel, l, pl.ds(0, LANES)]
                    for j in range(1, D // LANES):
                        acc = acc + e[j] * rows_ref[bufsel, l, pl.ds(j * LANES, LANES)]
                vec = jnp.where(lane == i, lanesum(acc), vec)
            return vec

        for c in range(2):
            copies[2 * c].wait()
            copies[2 * c + 1].wait()

            def g_body(g, _, c=c):
                off = c * LC + g * LANES
                out_v[pl.ds(off, LANES)] = (
                    out_v[pl.ds(off, LANES)] + dot16(rows_v, g * LANES, c))
                return 0
            lax.fori_loop(0, NG, g_body, 0)

        copies[4].wait()
        copies[5].wait()
        off = 2 * LC
        out_v[pl.ds(off, LANES)] = (
            out_v[pl.ds(off, LANES)] + dot16(rowst_v, 0, None))

        pltpu.sync_copy(out_v, out_hbm.at[b])
        return 0

    lax.fori_loop(0, BPW, row_body, 0)


@jax.jit
def _sparse_linear(embed, shortlist, weight, bias):
    mesh = plsc.VectorSubcoreMesh(
        core_axis_name="c", subcore_axis_name="s",
        num_cores=NC, num_subcores=NS)
    kfn = pl.kernel(
        _sc_body,
        out_type=jax.ShapeDtypeStruct((B, LPAD), jnp.float32),
        mesh=mesh,
        scratch_types=[
            pltpu.VMEM((D,), jnp.float32),
            pltpu.VMEM((L,), jnp.int32),
            pltpu.VMEM((2, LC, D), jnp.float32),
            pltpu.VMEM((LANES, D), jnp.float32),
            pltpu.VMEM((LPAD,), jnp.float32),
            pltpu.SemaphoreType.DMA((3,)),
        ],
    )
    return kfn(embed, shortlist, weight, bias)[:, :L]


def kernel(embed, shortlist, weight, bias):
    return _sparse_linear(embed, shortlist.astype(jnp.int32), weight,
                          bias.reshape(V))
